# baseline (speedup 1.0000x reference)
import functools

import numpy as np
import jax
import jax.numpy as jnp
from jax import lax
from jax.experimental import pallas as pl
from jax.experimental.pallas import tpu as pltpu
from jax.experimental.pallas import tpu_sc as plsc

_D = 256
_SEQ = 200
_NC, _NS, _L = 2, 16, 16
_NW = _NC * _NS
_CH = 40


def _positional_encoding() -> np.ndarray:
    depth_h = _D / 2
    positions = np.arange(_SEQ)[:, np.newaxis]
    depths = np.arange(depth_h)[np.newaxis, :] / depth_h
    angle_rates = 1 / 10000 ** depths
    angle_rads = positions * angle_rates
    return np.concatenate(
        [np.sin(angle_rads), np.cos(angle_rads)], axis=-1
    ).astype(np.float32)


_POS = _positional_encoding()


@functools.lru_cache(maxsize=None)
def _build(n_rows: int):
    assert n_rows % (_NW * _SEQ) == 0
    rpw = n_rows // _NW
    n_chunks = rpw // _CH
    chunks_per_seq = _SEQ // _CH

    mesh = plsc.VectorSubcoreMesh(
        core_axis_name="c", subcore_axis_name="s",
        num_cores=_NC, num_subcores=_NS,
    )

    @functools.partial(
        pl.kernel,
        out_type=jax.ShapeDtypeStruct((n_rows, _D), jnp.float32),
        mesh=mesh,
        scratch_types=[
            pltpu.VMEM((_SEQ, _D), jnp.float32),
            pltpu.VMEM((_CH,), jnp.int32),
            pltpu.VMEM((_CH, _D), jnp.float32),
            pltpu.SemaphoreType.DMA,
        ],
    )
    def embed(idx_hbm, table_hbm, pos_hbm, out_hbm, pos_v, idx_v, g_v, sem):
        wid = lax.axis_index("s") * _NC + lax.axis_index("c")
        base = wid * rpw
        pltpu.sync_copy(pos_hbm, pos_v)

        def chunk_body(c, carry):
            rowbase = base + c * _CH
            pltpu.sync_copy(idx_hbm.at[pl.ds(rowbase, _CH)], idx_v)
            pltpu.async_copy(table_hbm.at[idx_v], g_v, sem).wait()
            posrow = (c % chunks_per_seq) * _CH

            def row_body(r, carry2):
                for dsub in range(_D // _L):
                    off = dsub * _L
                    g_v[r, pl.ds(off, _L)] = (
                        g_v[r, pl.ds(off, _L)] * 16.0
                        + pos_v[posrow + r, pl.ds(off, _L)]
                    )
                return carry2

            lax.fori_loop(0, _CH, row_body, 0)
            pltpu.sync_copy(g_v, out_hbm.at[pl.ds(rowbase, _CH)])
            return carry

        lax.fori_loop(0, n_chunks, chunk_body, 0)

    return embed


def kernel(input, table):
    b, s = input.shape
    idx = input.reshape(-1).astype(jnp.int32)
    pos = jnp.asarray(_POS)
    out = _build(idx.shape[0])(idx, table, pos)
    return out.reshape(b, s, _D)


# speedup vs baseline: 1.6004x; 1.6004x over previous
import functools

import numpy as np
import jax
import jax.numpy as jnp
from jax import lax
from jax.experimental import pallas as pl
from jax.experimental.pallas import tpu as pltpu
from jax.experimental.pallas import tpu_sc as plsc

_D = 256
_SEQ = 200
_NC, _NS, _L = 2, 16, 16
_NW = _NC * _NS
_CH = 40
_NBUF = 4


def _positional_encoding() -> np.ndarray:
    depth_h = _D / 2
    positions = np.arange(_SEQ)[:, np.newaxis]
    depths = np.arange(depth_h)[np.newaxis, :] / depth_h
    angle_rates = 1 / 10000 ** depths
    angle_rads = positions * angle_rates
    return np.concatenate(
        [np.sin(angle_rads), np.cos(angle_rads)], axis=-1
    ).astype(np.float32)


_POS = _positional_encoding()


@functools.lru_cache(maxsize=None)
def _build(n_rows: int):
    assert n_rows % (_NW * _SEQ) == 0
    rpw = n_rows // _NW
    n_chunks = rpw // _CH
    assert n_chunks % _NBUF == 0
    chunks_per_seq = _SEQ // _CH

    mesh = plsc.VectorSubcoreMesh(
        core_axis_name="c", subcore_axis_name="s",
        num_cores=_NC, num_subcores=_NS,
    )

    @functools.partial(
        pl.kernel,
        out_type=jax.ShapeDtypeStruct((n_rows, _D), jnp.float32),
        mesh=mesh,
        scratch_types=[
            pltpu.VMEM((_SEQ, _D), jnp.float32),
            pltpu.VMEM((n_chunks, _CH), jnp.int32),
            [pltpu.VMEM((_CH, _D), jnp.float32)] * _NBUF,
            pltpu.SemaphoreType.DMA((_NBUF,)),
            pltpu.SemaphoreType.DMA((_NBUF,)),
        ],
    )
    def embed(idx_hbm, table_hbm, pos_hbm, out_hbm, pos_v, idx_v, bufs,
              gsem, osem):
        wid = lax.axis_index("s") * _NC + lax.axis_index("c")
        base = wid * rpw
        pltpu.sync_copy(idx_hbm.at[wid], idx_v)
        pltpu.sync_copy(pos_hbm, pos_v)

        def gather(c, b):
            pltpu.async_copy(table_hbm.at[idx_v.at[c]], bufs[b], gsem.at[b])

        def gather_wait(c, b):
            pltpu.make_async_copy(
                table_hbm.at[idx_v.at[c]], bufs[b], gsem.at[b]).wait()

        def out_slice(c):
            return out_hbm.at[pl.ds(base + c * _CH, _CH)]

        gather(0, 0)
        gather(1, 1)

        @pl.loop(0, n_chunks, step=_NBUF)
        def chunk_group(t):
            for b in range(_NBUF):
                c = t + b
                gather_wait(c, b)

                posrow = (c % chunks_per_seq) * _CH
                buf = bufs[b]

                @pl.loop(0, _CH)
                def row_body(r):
                    for dsub in range(_D // _L):
                        off = dsub * _L
                        buf[r, pl.ds(off, _L)] = (
                            buf[r, pl.ds(off, _L)] * 16.0
                            + pos_v[posrow + r, pl.ds(off, _L)]
                        )

                pltpu.async_copy(buf, out_slice(c), osem.at[b])

                b2 = (b + 2) % _NBUF
                c2 = c + 2

                @pl.when(c2 < n_chunks)
                def _():
                    @pl.when(c >= 2)
                    def _():
                        pltpu.make_async_copy(
                            bufs[b2], out_slice(c2 - _NBUF), osem.at[b2]
                        ).wait()
                    gather(c2, b2)

        for b in range(_NBUF):
            c = n_chunks - _NBUF + b
            pltpu.make_async_copy(bufs[b], out_slice(c), osem.at[b]).wait()

    return embed


def kernel(input, table):
    b, s = input.shape
    n_rows = b * s
    rpw = n_rows // _NW
    idx = input.reshape(_NW, rpw // _CH, _CH).astype(jnp.int32)
    pos = jnp.asarray(_POS)
    out = _build(n_rows)(idx, table, pos)
    return out.reshape(b, s, _D)


# speedup vs baseline: 2.5748x; 1.6088x over previous
import functools

import numpy as np
import jax
import jax.numpy as jnp
from jax import lax
from jax.experimental import pallas as pl
from jax.experimental.pallas import tpu as pltpu
from jax.experimental.pallas import tpu_sc as plsc

_D = 256
_SEQ = 200
_B = 1024
_NC, _NS, _L = 2, 16, 16
_NW = _NC * _NS
_PG = 8
_BG = 4
_PPW = _SEQ // _PG
_BPW = _B // _BG
_CH = 64
_QS = _BPW // _CH
_NCHUNK = _PPW * _QS
_NBUF = 4


def _positional_encoding() -> np.ndarray:
    depth_h = _D / 2
    positions = np.arange(_SEQ)[:, np.newaxis]
    depths = np.arange(depth_h)[np.newaxis, :] / depth_h
    angle_rates = 1 / 10000 ** depths
    angle_rads = positions * angle_rates
    return np.concatenate(
        [np.sin(angle_rads), np.cos(angle_rads)], axis=-1
    ).astype(np.float32)


_POS = _positional_encoding()


def _build():
    mesh = plsc.VectorSubcoreMesh(
        core_axis_name="c", subcore_axis_name="s",
        num_cores=_NC, num_subcores=_NS,
    )

    @functools.partial(
        pl.kernel,
        out_type=jax.ShapeDtypeStruct((_B, _SEQ * _D), jnp.float32),
        mesh=mesh,
        scratch_types=[
            pltpu.VMEM((_PPW * _D,), jnp.float32),
            pltpu.VMEM((_NCHUNK * _CH,), jnp.int32),
            [pltpu.VMEM((_CH, _D), jnp.float32)] * _NBUF,
            pltpu.SemaphoreType.DMA((_NBUF,)),
            pltpu.SemaphoreType.DMA((_NBUF,)),
        ],
    )
    def embed(idx_hbm, table_hbm, pos_hbm, out_hbm, pos_v, idx_v, bufs,
              gsem, osem):
        wid = lax.axis_index("s") * _NC + lax.axis_index("c")
        pg = wid // _BG
        bg = wid % _BG
        pltpu.sync_copy(idx_hbm.at[wid], idx_v)
        pltpu.sync_copy(pos_hbm.at[pl.ds(pg * _PPW * _D, _PPW * _D)], pos_v)

        def idx_slice(c):
            return idx_v.at[pl.ds(c * _CH, _CH)]

        def gather(c, b):
            pltpu.async_copy(table_hbm.at[idx_slice(c)], bufs[b], gsem.at[b])

        def gather_wait(c, b):
            pltpu.make_async_copy(
                table_hbm.at[idx_slice(c)], bufs[b], gsem.at[b]).wait()

        def out_slice(c):
            p = pg * _PPW + c // _QS
            b0 = bg * _BPW + (c % _QS) * _CH
            return out_hbm.at[pl.ds(b0, _CH), pl.ds(p * _D, _D)]

        gather(0, 0)
        gather(1, 1)

        @pl.loop(0, _NCHUNK, step=_NBUF)
        def chunk_group(t):
            for b in range(_NBUF):
                c = t + b
                gather_wait(c, b)

                buf = bufs[b]
                p_loc = c // _QS
                pvs = [pos_v[pl.ds(p_loc * _D + k * _L, _L)]
                       for k in range(_D // _L)]

                @pl.loop(0, _CH)
                def row_body(r):
                    for k in range(_D // _L):
                        off = k * _L
                        buf[r, pl.ds(off, _L)] = (
                            buf[r, pl.ds(off, _L)] * 16.0 + pvs[k]
                        )

                pltpu.async_copy(buf, out_slice(c), osem.at[b])

                b2 = (b + 2) % _NBUF
                c2 = c + 2

                @pl.when(c2 < _NCHUNK)
                def _():
                    @pl.when(c >= 2)
                    def _():
                        pltpu.make_async_copy(
                            bufs[b2], out_slice(c2 - _NBUF), osem.at[b2]
                        ).wait()
                    gather(c2, b2)

        for b in range(_NBUF):
            c = _NCHUNK - _NBUF + b
            pltpu.make_async_copy(bufs[b], out_slice(c), osem.at[b]).wait()

    return embed


def kernel(input, table):
    idx = jnp.transpose(input.astype(jnp.int32))
    idx = idx.reshape(_PG, _PPW, _BG, _QS, _CH)
    idx = jnp.transpose(idx, (0, 2, 1, 3, 4))
    idx = idx.reshape(_NW, _NCHUNK * _CH)
    pos = jnp.asarray(_POS).reshape(-1)
    out = _build()(idx, table, pos)
    return out.reshape(_B, _SEQ, _D)


# speedup vs baseline: 3.8541x; 1.4969x over previous
import functools

import numpy as np
import jax
import jax.numpy as jnp
from jax import lax
from jax.experimental import pallas as pl
from jax.experimental.pallas import tpu as pltpu
from jax.experimental.pallas import tpu_sc as plsc

_D = 256
_SEQ = 200
_B = 1024
_NC, _NS, _L = 2, 16, 16
_NW = _NC * _NS
_PB = 8
_BB = 8
_NPB = _SEQ // _PB
_NBB = _B // _BB
_CH = _PB * _BB
_NCHUNK = _NPB * _NBB // _NW
_NBUF = 4
_POSPAD = _NPB + 1


def _positional_encoding() -> np.ndarray:
    depth_h = _D / 2
    positions = np.arange(_SEQ)[:, np.newaxis]
    depths = np.arange(depth_h)[np.newaxis, :] / depth_h
    angle_rates = 1 / 10000 ** depths
    angle_rads = positions * angle_rates
    return np.concatenate(
        [np.sin(angle_rads), np.cos(angle_rads)], axis=-1
    ).astype(np.float32)


_POS = np.zeros((_POSPAD * _PB, _D), np.float32)
_POS[:_SEQ] = _positional_encoding()


def _build():
    mesh = plsc.VectorSubcoreMesh(
        core_axis_name="c", subcore_axis_name="s",
        num_cores=_NC, num_subcores=_NS,
    )

    @functools.partial(
        pl.kernel,
        out_type=jax.ShapeDtypeStruct((_NBB, _BB, _SEQ, _D), jnp.float32),
        mesh=mesh,
        scratch_types=[
            pltpu.VMEM((2 * _PB * _D,), jnp.float32),
            pltpu.VMEM((_NCHUNK * _CH,), jnp.int32),
            [pltpu.VMEM((_CH, _D), jnp.float32)] * _NBUF,
            pltpu.SemaphoreType.DMA((_NBUF,)),
            pltpu.SemaphoreType.DMA((_NBUF,)),
        ],
    )
    def embed(idx_hbm, table_hbm, pos_hbm, out_hbm, pos_v, idx_v, bufs,
              gsem, osem):
        wid = lax.axis_index("s") * _NC + lax.axis_index("c")
        u0 = wid * _NCHUNK
        pblk0 = u0 // _NBB
        pltpu.sync_copy(idx_hbm.at[wid], idx_v)
        pltpu.sync_copy(
            pos_hbm.at[pl.ds(pblk0 * _PB * _D, 2 * _PB * _D)], pos_v)

        def gather(c, b):
            pltpu.async_copy(
                table_hbm.at[idx_v.at[pl.ds(c * _CH, _CH)]],
                bufs[b], gsem.at[b])

        def gather_wait(c, b):
            pltpu.make_async_copy(
                table_hbm.at[idx_v.at[pl.ds(c * _CH, _CH)]],
                bufs[b], gsem.at[b]).wait()

        def writeout(c, b):
            u = u0 + c
            bblk = u % _NBB
            p0 = (u // _NBB) * _PB
            for b_sub in range(_BB):
                pltpu.async_copy(
                    bufs[b].at[pl.ds(b_sub * _PB, _PB)],
                    out_hbm.at[bblk, b_sub, pl.ds(p0, _PB)],
                    osem.at[b])

        def writeout_wait(b):
            pltpu.make_async_copy(table_hbm.at[pl.ds(0, _CH)], bufs[b],
                                  osem.at[b]).wait()

        gather(0, 0)
        gather(1, 1)

        @pl.loop(0, _NCHUNK, step=_NBUF)
        def chunk_group(t):
            for b in range(_NBUF):
                c = t + b
                gather_wait(c, b)

                buf = bufs[b]
                u = u0 + c
                ploc0 = (u // _NBB - pblk0) * _PB
                for p_sub in range(_PB):
                    pvs = [pos_v[pl.ds((ploc0 + p_sub) * _D + k * _L, _L)]
                           for k in range(_D // _L)]

                    @pl.loop(0, _BB)
                    def row_body(b_sub):
                        r = b_sub * _PB + p_sub
                        for k in range(_D // _L):
                            off = k * _L
                            buf[r, pl.ds(off, _L)] = (
                                buf[r, pl.ds(off, _L)] * 16.0 + pvs[k]
                            )

                writeout(c, b)

                b2 = (b + 2) % _NBUF
                c2 = c + 2

                @pl.when(c2 < _NCHUNK)
                def _():
                    @pl.when(c >= 2)
                    def _():
                        writeout_wait(b2)
                    gather(c2, b2)

        for b in range(_NBUF):
            writeout_wait(b)

    return embed


def kernel(input, table):
    idx = input.astype(jnp.int32)
    idx = idx.reshape(_NBB, _BB, _NPB, _PB)
    idx = jnp.transpose(idx, (2, 0, 1, 3))
    idx = idx.reshape(_NW, _NCHUNK * _CH)
    pos = jnp.asarray(_POS).reshape(-1)
    out = _build()(idx, table, pos)
    return out.reshape(_B, _SEQ, _D)
